# baseline (speedup 1.0000x reference)
import functools

import jax
import jax.numpy as jnp
from jax import lax
from jax.experimental import pallas as pl
from jax.experimental.pallas import tpu as pltpu
from jax.experimental.pallas import tpu_sc as plsc

B = 8
N = 20000
G = 50
L = 16
NT = 32
CHUNKS = B * N // L
BASE = CHUNKS // NT
EXTRA = CHUNKS - BASE * NT
CPB = N // L
EPS = 1e-7
IOU_THR = 0.1
LN2 = 0.6931471805599453
UNROLL = 50


def _logf(x):
    bits = lax.bitcast_convert_type(x, jnp.int32)
    e = lax.shift_right_logical(bits, 23) - 127
    m = lax.bitcast_convert_type((bits & 0x007FFFFF) | 0x3F800000, jnp.float32)
    big = m > 1.4142135381698608
    m = jnp.where(big, m * 0.5, m)
    e = e + big.astype(jnp.int32)
    t = (m - 1.0) / (m + 1.0)
    t2 = t * t
    p = 1.0 + t2 * (1 / 3 + t2 * (1 / 5 + t2 * (1 / 7)))
    return e.astype(jnp.float32) * LN2 + 2.0 * t * p


def _body(preds_hbm, gt_hbm, out_hbm, stage, gtraw,
          cnx, cny, cnw, cnh, cx2, cy2, car, accv):
    wid = lax.axis_index("s") * 2 + lax.axis_index("c")
    iota = lax.iota(jnp.int32, L)
    col0 = jnp.zeros((L,), jnp.int32)
    col1 = col0 + 1
    col2 = col0 + 2
    col3 = col0 + 3
    col4 = col0 + 4

    pltpu.sync_copy(preds_hbm.at[pl.ds(wid * (BASE * L), BASE * L), :],
                    stage.at[pl.ds(0, BASE * L), :])

    @pl.when(wid < EXTRA)
    def _():
        pltpu.sync_copy(
            preds_hbm.at[pl.ds((BASE * NT + wid) * L, L), :],
            stage.at[pl.ds(BASE * L, L), :])

    pltpu.sync_copy(gt_hbm, gtraw)

    for j in range(B * G // L):
        rows = iota + j * L
        gx = plsc.load_gather(gtraw, [rows, col0])
        gy = plsc.load_gather(gtraw, [rows, col1])
        gw = plsc.load_gather(gtraw, [rows, col2])
        gh = plsc.load_gather(gtraw, [rows, col3])
        nx = gx / 512.0
        ny = gy / 512.0
        nw = gw / 512.0
        nh = gh / 512.0
        x2 = nx + nw
        y2 = ny + nh
        s = pl.ds(j * L, L)
        cnx[s] = nx
        cny[s] = ny
        cnw[s] = nw
        cnh[s] = nh
        cx2[s] = x2
        cy2[s] = y2
        car[s] = (x2 - nx) * (y2 - ny)

    zero = jnp.zeros((L,), jnp.float32)
    def chunk_compute(c):
        gc = jnp.where(c < BASE, wid * BASE + c, BASE * NT + wid)
        gtoff = (gc // CPB) * G
        prow = iota + c * L
        px = plsc.load_gather(stage, [prow, col0])
        py = plsc.load_gather(stage, [prow, col1])
        pw = plsc.load_gather(stage, [prow, col2])
        ph = plsc.load_gather(stage, [prow, col3])
        pcf = plsc.load_gather(stage, [prow, col4])
        ax2 = px + pw
        ay2 = py + ph
        area_a = (ax2 - px) * (ay2 - py)

        def g_step(g, bi, bg):
            gidx = jnp.full((L,), gtoff + g, jnp.int32)
            bx1 = plsc.load_gather(cnx, [gidx])
            by1 = plsc.load_gather(cny, [gidx])
            bx2 = plsc.load_gather(cx2, [gidx])
            by2 = plsc.load_gather(cy2, [gidx])
            ab = plsc.load_gather(car, [gidx])
            ix = jnp.maximum(jnp.minimum(ax2, bx2) - jnp.maximum(px, bx1), 0.0)
            iy = jnp.maximum(jnp.minimum(ay2, by2) - jnp.maximum(py, by1), 0.0)
            inter = ix * iy
            union = area_a + ab - inter
            iou = inter / (union + EPS)
            upd = iou > bi
            bi = jnp.where(upd, iou, bi)
            bg = jnp.where(upd, gidx, bg)
            return bi, bg

        best_iou = zero - 1.0
        best_g = jnp.zeros((L,), jnp.int32)
        for g in range(G):
            best_iou, best_g = g_step(g, best_iou, best_g)

        mf = (best_iou >= IOU_THR).astype(jnp.float32)
        tx = plsc.load_gather(cnx, [best_g])
        ty = plsc.load_gather(cny, [best_g])
        tw = plsc.load_gather(cnw, [best_g])
        th = plsc.load_gather(cnh, [best_g])
        s = zero
        for p, t in ((px, tx), (py, ty), (pw, tw), (ph, th)):
            d = p - t
            ad = jnp.abs(d)
            s = s + jnp.where(ad < 1.0, 0.5 * d * d, ad - 0.5)
        pcl = jnp.clip(pcf, EPS, 1.0 - EPS)
        bce = -(mf * _logf(pcl) + (1.0 - mf) * _logf(1.0 - pcl))
        return mf * s, bce, mf

    nchunks = jnp.where(wid < EXTRA, BASE + 1, BASE)

    def chunk_body(c, carry):
        bbox_a, conf_a, match_a = carry
        bc, cc, mc = chunk_compute(c)
        return bbox_a + bc, conf_a + cc, match_a + mc

    bbox_a, conf_a, match_a = lax.fori_loop(
        0, nchunks, chunk_body, (zero, zero, zero))

    accv[0] = bbox_a
    accv[1] = conf_a
    accv[2] = match_a
    pltpu.sync_copy(accv, out_hbm.at[wid])


_sc_call = functools.partial(
    pl.kernel,
    mesh=plsc.VectorSubcoreMesh(core_axis_name="c", subcore_axis_name="s"),
    out_type=jax.ShapeDtypeStruct((NT, 3, L), jnp.float32),
    compiler_params=pltpu.CompilerParams(
        needs_layout_passes=False, use_tc_tiling_on_sc=False),
    scratch_types=[
        pltpu.VMEM(((BASE + 1) * L, 5), jnp.float32),
        pltpu.VMEM((B * G, 4), jnp.float32),
        pltpu.VMEM((B * G,), jnp.float32),
        pltpu.VMEM((B * G,), jnp.float32),
        pltpu.VMEM((B * G,), jnp.float32),
        pltpu.VMEM((B * G,), jnp.float32),
        pltpu.VMEM((B * G,), jnp.float32),
        pltpu.VMEM((B * G,), jnp.float32),
        pltpu.VMEM((B * G,), jnp.float32),
        pltpu.VMEM((3, L), jnp.float32),
    ],
)(_body)


def kernel(preds, images, gt_boxes):
    del images
    parts = _sc_call(preds.reshape(B * N, 5), gt_boxes.reshape(B * G, 4))
    sums = jnp.sum(parts, axis=(0, 2))
    tb, tc, tm = sums[0], sums[1], sums[2]
    total_bbox = jnp.where(tm > 0, tb / jnp.maximum(tm, 1.0), 0.0)
    total_conf = tc / (B * N)
    total_loss = total_bbox + total_conf
    return (total_loss, total_bbox, total_conf)


# speedup vs baseline: 2.2819x; 2.2819x over previous
import functools

import jax
import jax.numpy as jnp
from jax import lax
from jax.experimental import pallas as pl
from jax.experimental.pallas import tpu as pltpu
from jax.experimental.pallas import tpu_sc as plsc

B = 8
N = 20000
G = 50
L = 16
NT = 32
CHUNKS = B * N // L
BASE = CHUNKS // NT
EXTRA = CHUNKS - BASE * NT
CPB = N // L
EPS = 1e-7
IOU_THR = 0.1
LN2 = 0.6931471805599453
UNROLL = 50


def _logf(x):
    bits = lax.bitcast_convert_type(x, jnp.int32)
    e = lax.shift_right_logical(bits, 23) - 127
    m = lax.bitcast_convert_type((bits & 0x007FFFFF) | 0x3F800000, jnp.float32)
    big = m > 1.4142135381698608
    m = jnp.where(big, m * 0.5, m)
    e = e + big.astype(jnp.int32)
    t = (m - 1.0) / (m + 1.0)
    t2 = t * t
    p = 1.0 + t2 * (1 / 3 + t2 * (1 / 5 + t2 * (1 / 7)))
    return e.astype(jnp.float32) * LN2 + 2.0 * t * p


def _body(preds_hbm, gt_hbm, out_hbm, sx, sy, sw, sh, scf, gtraw,
          cnx, cny, cnw, cnh, cx2, cy2, car, accv):
    wid = lax.axis_index("s") * 2 + lax.axis_index("c")
    iota = lax.iota(jnp.int32, L)

    for comp, ref in enumerate((sx, sy, sw, sh, scf)):
        pltpu.sync_copy(
            preds_hbm.at[pl.ds(comp * (B * N) + wid * (BASE * L), BASE * L)],
            ref.at[pl.ds(0, BASE * L)])

    @pl.when(wid < EXTRA)
    def _():
        for comp, ref in enumerate((sx, sy, sw, sh, scf)):
            pltpu.sync_copy(
                preds_hbm.at[pl.ds(comp * (B * N) + (BASE * NT + wid) * L, L)],
                ref.at[pl.ds(BASE * L, L)])

    pltpu.sync_copy(gt_hbm, gtraw)

    for j in range(B * G // L):
        idx4 = (iota + j * L) * 4
        gx = plsc.load_gather(gtraw, [idx4])
        gy = plsc.load_gather(gtraw, [idx4 + 1])
        gw = plsc.load_gather(gtraw, [idx4 + 2])
        gh = plsc.load_gather(gtraw, [idx4 + 3])
        nx = gx / 512.0
        ny = gy / 512.0
        nw = gw / 512.0
        nh = gh / 512.0
        x2 = nx + nw
        y2 = ny + nh
        s = pl.ds(j * L, L)
        cnx[s] = nx
        cny[s] = ny
        cnw[s] = nw
        cnh[s] = nh
        cx2[s] = x2
        cy2[s] = y2
        car[s] = (x2 - nx) * (y2 - ny)

    zero = jnp.zeros((L,), jnp.float32)
    def chunk_compute(c):
        gc = jnp.where(c < BASE, wid * BASE + c, BASE * NT + wid)
        gtoff = (gc // CPB) * G
        prow = pl.ds(c * L, L)
        px = sx[prow]
        py = sy[prow]
        pw = sw[prow]
        ph = sh[prow]
        pcf = scf[prow]
        ax2 = px + pw
        ay2 = py + ph
        area_a = (ax2 - px) * (ay2 - py)

        def g_step(g, bi, bg):
            gidx = jnp.full((L,), gtoff + g, jnp.int32)
            bx1 = plsc.load_gather(cnx, [gidx])
            by1 = plsc.load_gather(cny, [gidx])
            bx2 = plsc.load_gather(cx2, [gidx])
            by2 = plsc.load_gather(cy2, [gidx])
            ab = plsc.load_gather(car, [gidx])
            ix = jnp.maximum(jnp.minimum(ax2, bx2) - jnp.maximum(px, bx1), 0.0)
            iy = jnp.maximum(jnp.minimum(ay2, by2) - jnp.maximum(py, by1), 0.0)
            inter = ix * iy
            union = area_a + ab - inter
            iou = inter / (union + EPS)
            upd = iou > bi
            bi = jnp.where(upd, iou, bi)
            bg = jnp.where(upd, gidx, bg)
            return bi, bg

        best_iou = zero - 1.0
        best_g = jnp.zeros((L,), jnp.int32)
        for g in range(G):
            best_iou, best_g = g_step(g, best_iou, best_g)

        mf = (best_iou >= IOU_THR).astype(jnp.float32)
        tx = plsc.load_gather(cnx, [best_g])
        ty = plsc.load_gather(cny, [best_g])
        tw = plsc.load_gather(cnw, [best_g])
        th = plsc.load_gather(cnh, [best_g])
        s = zero
        for p, t in ((px, tx), (py, ty), (pw, tw), (ph, th)):
            d = p - t
            ad = jnp.abs(d)
            s = s + jnp.where(ad < 1.0, 0.5 * d * d, ad - 0.5)
        pcl = jnp.clip(pcf, EPS, 1.0 - EPS)
        bce = -(mf * _logf(pcl) + (1.0 - mf) * _logf(1.0 - pcl))
        return mf * s, bce, mf

    nchunks = jnp.where(wid < EXTRA, BASE + 1, BASE)

    def chunk_body(c, carry):
        bbox_a, conf_a, match_a = carry
        bc, cc, mc = chunk_compute(c)
        return bbox_a + bc, conf_a + cc, match_a + mc

    bbox_a, conf_a, match_a = lax.fori_loop(
        0, nchunks, chunk_body, (zero, zero, zero))

    accv[0] = bbox_a
    accv[1] = conf_a
    accv[2] = match_a
    pltpu.sync_copy(accv, out_hbm.at[wid])


_sc_call = functools.partial(
    pl.kernel,
    mesh=plsc.VectorSubcoreMesh(core_axis_name="c", subcore_axis_name="s"),
    out_type=jax.ShapeDtypeStruct((NT, 3, L), jnp.float32),
    compiler_params=pltpu.CompilerParams(needs_layout_passes=False),
    scratch_types=[
        pltpu.VMEM(((BASE + 1) * L,), jnp.float32),
        pltpu.VMEM(((BASE + 1) * L,), jnp.float32),
        pltpu.VMEM(((BASE + 1) * L,), jnp.float32),
        pltpu.VMEM(((BASE + 1) * L,), jnp.float32),
        pltpu.VMEM(((BASE + 1) * L,), jnp.float32),
        pltpu.VMEM((B * G * 4,), jnp.float32),
        pltpu.VMEM((B * G,), jnp.float32),
        pltpu.VMEM((B * G,), jnp.float32),
        pltpu.VMEM((B * G,), jnp.float32),
        pltpu.VMEM((B * G,), jnp.float32),
        pltpu.VMEM((B * G,), jnp.float32),
        pltpu.VMEM((B * G,), jnp.float32),
        pltpu.VMEM((B * G,), jnp.float32),
        pltpu.VMEM((3, L), jnp.float32),
    ],
)(_body)


def kernel(preds, images, gt_boxes):
    del images
    pf = jnp.transpose(preds.reshape(B * N, 5)).reshape(-1)
    parts = _sc_call(pf, gt_boxes.reshape(-1))
    sums = jnp.sum(parts, axis=(0, 2))
    tb, tc, tm = sums[0], sums[1], sums[2]
    total_bbox = jnp.where(tm > 0, tb / jnp.maximum(tm, 1.0), 0.0)
    total_conf = tc / (B * N)
    total_loss = total_bbox + total_conf
    return (total_loss, total_bbox, total_conf)


# speedup vs baseline: 3.2878x; 1.4408x over previous
import functools

import jax
import jax.numpy as jnp
from jax import lax
from jax.experimental import pallas as pl
from jax.experimental.pallas import tpu as pltpu
from jax.experimental.pallas import tpu_sc as plsc

B = 8
N = 20000
G = 50
L = 16
NT = 32
NB_SC = 4
TCB = B - NB_SC
SC_CHUNKS = NB_SC * N // L
BASE = SC_CHUNKS // NT
EXTRA = SC_CHUNKS - BASE * NT
CPB = N // L
NP = 20480
SUB = 2048
NSUB = NP // SUB
EPS = 1e-7
IOU_THR = 0.1
LN2 = 0.6931471805599453


def _logf(x):
    bits = lax.bitcast_convert_type(x, jnp.int32)
    e = lax.shift_right_logical(bits, 23) - 127
    m = lax.bitcast_convert_type((bits & 0x007FFFFF) | 0x3F800000, jnp.float32)
    big = m > 1.4142135381698608
    m = jnp.where(big, m * 0.5, m)
    e = e + big.astype(jnp.int32)
    t = (m - 1.0) / (m + 1.0)
    t2 = t * t
    p = 1.0 + t2 * (1 / 3 + t2 * (1 / 5 + t2 * (1 / 7)))
    return e.astype(jnp.float32) * LN2 + 2.0 * t * p


def _sc_body(preds_hbm, gt_hbm, out_hbm, sx, sy, sw, sh, scf, gtraw,
             cnx, cny, cnw, cnh, cx2, cy2, car, accv):
    wid = lax.axis_index("s") * 2 + lax.axis_index("c")
    iota = lax.iota(jnp.int32, L)

    for comp, ref in enumerate((sx, sy, sw, sh, scf)):
        pltpu.sync_copy(
            preds_hbm.at[pl.ds(comp * (B * N) + wid * (BASE * L), BASE * L)],
            ref.at[pl.ds(0, BASE * L)])

    @pl.when(wid < EXTRA)
    def _():
        for comp, ref in enumerate((sx, sy, sw, sh, scf)):
            pltpu.sync_copy(
                preds_hbm.at[pl.ds(comp * (B * N) + (BASE * NT + wid) * L, L)],
                ref.at[pl.ds(BASE * L, L)])

    pltpu.sync_copy(gt_hbm, gtraw)

    for j in range(B * G // L):
        idx4 = (iota + j * L) * 4
        gx = plsc.load_gather(gtraw, [idx4])
        gy = plsc.load_gather(gtraw, [idx4 + 1])
        gw = plsc.load_gather(gtraw, [idx4 + 2])
        gh = plsc.load_gather(gtraw, [idx4 + 3])
        nx = gx / 512.0
        ny = gy / 512.0
        nw = gw / 512.0
        nh = gh / 512.0
        x2 = nx + nw
        y2 = ny + nh
        s = pl.ds(j * L, L)
        cnx[s] = nx
        cny[s] = ny
        cnw[s] = nw
        cnh[s] = nh
        cx2[s] = x2
        cy2[s] = y2
        car[s] = (x2 - nx) * (y2 - ny)

    zero = jnp.zeros((L,), jnp.float32)

    def chunk_compute(c):
        gc = jnp.where(c < BASE, wid * BASE + c, BASE * NT + wid)
        gtoff = (gc // CPB) * G
        prow = pl.ds(c * L, L)
        px = sx[prow]
        py = sy[prow]
        pw = sw[prow]
        ph = sh[prow]
        pcf = scf[prow]
        ax2 = px + pw
        ay2 = py + ph
        area_a = (ax2 - px) * (ay2 - py)

        def g_step(g, bi, bg):
            gidx = jnp.full((L,), gtoff + g, jnp.int32)
            bx1 = plsc.load_gather(cnx, [gidx])
            by1 = plsc.load_gather(cny, [gidx])
            bx2 = plsc.load_gather(cx2, [gidx])
            by2 = plsc.load_gather(cy2, [gidx])
            ab = plsc.load_gather(car, [gidx])
            ix = jnp.maximum(jnp.minimum(ax2, bx2) - jnp.maximum(px, bx1), 0.0)
            iy = jnp.maximum(jnp.minimum(ay2, by2) - jnp.maximum(py, by1), 0.0)
            inter = ix * iy
            union = area_a + ab - inter
            iou = inter / (union + EPS)
            upd = iou > bi
            bi = jnp.where(upd, iou, bi)
            bg = jnp.where(upd, gidx, bg)
            return bi, bg

        best_iou = zero - 1.0
        best_g = jnp.zeros((L,), jnp.int32)
        for g in range(G):
            best_iou, best_g = g_step(g, best_iou, best_g)

        mf = (best_iou >= IOU_THR).astype(jnp.float32)
        tx = plsc.load_gather(cnx, [best_g])
        ty = plsc.load_gather(cny, [best_g])
        tw = plsc.load_gather(cnw, [best_g])
        th = plsc.load_gather(cnh, [best_g])
        s = zero
        for p, t in ((px, tx), (py, ty), (pw, tw), (ph, th)):
            d = p - t
            ad = jnp.abs(d)
            s = s + jnp.where(ad < 1.0, 0.5 * d * d, ad - 0.5)
        pcl = jnp.clip(pcf, EPS, 1.0 - EPS)
        bce = -(mf * _logf(pcl) + (1.0 - mf) * _logf(1.0 - pcl))
        return mf * s, bce, mf

    nchunks = jnp.where(wid < EXTRA, BASE + 1, BASE)

    def chunk_body(c, carry):
        bbox_a, conf_a, match_a = carry
        bc, cc, mc = chunk_compute(c)
        return bbox_a + bc, conf_a + cc, match_a + mc

    bbox_a, conf_a, match_a = lax.fori_loop(
        0, nchunks, chunk_body, (zero, zero, zero))

    accv[0] = bbox_a
    accv[1] = conf_a
    accv[2] = match_a
    pltpu.sync_copy(accv, out_hbm.at[wid])


_sc_call = functools.partial(
    pl.kernel,
    mesh=plsc.VectorSubcoreMesh(core_axis_name="c", subcore_axis_name="s"),
    out_type=jax.ShapeDtypeStruct((NT, 3, L), jnp.float32),
    compiler_params=pltpu.CompilerParams(needs_layout_passes=False),
    scratch_types=[
        pltpu.VMEM(((BASE + 1) * L,), jnp.float32),
        pltpu.VMEM(((BASE + 1) * L,), jnp.float32),
        pltpu.VMEM(((BASE + 1) * L,), jnp.float32),
        pltpu.VMEM(((BASE + 1) * L,), jnp.float32),
        pltpu.VMEM(((BASE + 1) * L,), jnp.float32),
        pltpu.VMEM((B * G * 4,), jnp.float32),
        pltpu.VMEM((B * G,), jnp.float32),
        pltpu.VMEM((B * G,), jnp.float32),
        pltpu.VMEM((B * G,), jnp.float32),
        pltpu.VMEM((B * G,), jnp.float32),
        pltpu.VMEM((B * G,), jnp.float32),
        pltpu.VMEM((B * G,), jnp.float32),
        pltpu.VMEM((B * G,), jnp.float32),
        pltpu.VMEM((3, L), jnp.float32),
    ],
)(_sc_body)


def _tc_body(cols_ref, pf_ref, o_ref):
    b = pl.program_id(0)
    gbase = (NB_SC + b) * G
    rbase = b * NP
    CSTRIDE = TCB * NP
    iota = lax.iota(jnp.int32, SUB)

    def sub_body(j, carry):
        sb, sc_, sm = carry
        off = rbase + j * SUB
        px = pf_ref[pl.ds(off, SUB)]
        py = pf_ref[pl.ds(CSTRIDE + off, SUB)]
        pw = pf_ref[pl.ds(2 * CSTRIDE + off, SUB)]
        ph = pf_ref[pl.ds(3 * CSTRIDE + off, SUB)]
        pcf = pf_ref[pl.ds(4 * CSTRIDE + off, SUB)]
        valid = ((j * SUB + iota) < N).astype(jnp.float32)
        ax2 = px + pw
        ay2 = py + ph
        area_a = (ax2 - px) * (ay2 - py)
        bi = jnp.full((SUB,), -1.0, jnp.float32)
        tx = jnp.zeros((SUB,), jnp.float32)
        ty = tx
        tw = tx
        th = tx
        for g in range(G):
            nxs = cols_ref[0, gbase + g]
            nys = cols_ref[1, gbase + g]
            nws = cols_ref[2, gbase + g]
            nhs = cols_ref[3, gbase + g]
            x2s = cols_ref[4, gbase + g]
            y2s = cols_ref[5, gbase + g]
            abs_ = cols_ref[6, gbase + g]
            ix = jnp.maximum(jnp.minimum(ax2, x2s) - jnp.maximum(px, nxs), 0.0)
            iy = jnp.maximum(jnp.minimum(ay2, y2s) - jnp.maximum(py, nys), 0.0)
            inter = ix * iy
            union = area_a + abs_ - inter
            iou = inter / (union + EPS)
            upd = iou > bi
            bi = jnp.where(upd, iou, bi)
            tx = jnp.where(upd, nxs, tx)
            ty = jnp.where(upd, nys, ty)
            tw = jnp.where(upd, nws, tw)
            th = jnp.where(upd, nhs, th)
        mf = (bi >= IOU_THR).astype(jnp.float32) * valid
        s = jnp.zeros((SUB,), jnp.float32)
        for p, t in ((px, tx), (py, ty), (pw, tw), (ph, th)):
            d = p - t
            ad = jnp.abs(d)
            s = s + jnp.where(ad < 1.0, 0.5 * d * d, ad - 0.5)
        pcl = jnp.clip(pcf, EPS, 1.0 - EPS)
        bce = -(mf * jnp.log(pcl) + (1.0 - mf) * valid * jnp.log(1.0 - pcl))
        return sb + jnp.sum(mf * s), sc_ + jnp.sum(bce), sm + jnp.sum(mf)

    z = jnp.float32(0.0)
    sb, sc_, sm = lax.fori_loop(0, NSUB, sub_body, (z, z, z))
    lanes = lax.broadcasted_iota(jnp.int32, (1, 8, 128), 2)
    o_ref[...] = jnp.where(
        lanes == 0, sb,
        jnp.where(lanes == 1, sc_, jnp.where(lanes == 2, sm, 0.0)))


_tc_call = pl.pallas_call(
    _tc_body,
    grid=(TCB,),
    in_specs=[
        pl.BlockSpec(memory_space=pltpu.MemorySpace.SMEM),
        pl.BlockSpec((5 * TCB * NP,), lambda b: (0,)),
    ],
    out_specs=pl.BlockSpec((1, 8, 128), lambda b: (b, 0, 0)),
    out_shape=jax.ShapeDtypeStruct((TCB, 8, 128), jnp.float32),
)


def kernel(preds, images, gt_boxes):
    del images
    pf = jnp.transpose(preds.reshape(B * N, 5)).reshape(-1)
    gf = gt_boxes.reshape(-1)
    parts = _sc_call(pf, gf)

    gtn = gt_boxes / jnp.float32(512.0)
    nx = gtn[..., 0]
    ny = gtn[..., 1]
    nw = gtn[..., 2]
    nh = gtn[..., 3]
    x2 = nx + nw
    y2 = ny + nh
    ar = (x2 - nx) * (y2 - ny)
    cols = jnp.stack([nx, ny, nw, nh, x2, y2, ar]).reshape(7, B * G)
    ptc = jnp.transpose(preds[NB_SC:].reshape(TCB * N, 5)).reshape(5, TCB, N)
    ptc = jnp.pad(ptc, ((0, 0), (0, 0), (0, NP - N))).reshape(-1)
    o = _tc_call(cols, ptc)

    sums = jnp.sum(parts, axis=(0, 2))
    tb = sums[0] + jnp.sum(o[:, 0, 0])
    tcf = sums[1] + jnp.sum(o[:, 0, 1])
    tm = sums[2] + jnp.sum(o[:, 0, 2])
    total_bbox = jnp.where(tm > 0, tb / jnp.maximum(tm, 1.0), 0.0)
    total_conf = tcf / (B * N)
    total_loss = total_bbox + total_conf
    return (total_loss, total_bbox, total_conf)
